# baseline (speedup 1.0000x reference)
import functools

import jax
import jax.numpy as jnp
from jax import lax
from jax.experimental import pallas as pl
from jax.experimental.pallas import tpu as pltpu
from jax.experimental.pallas import tpu_sc as plsc

B = 16384
C = 100
S_SCALE = 30.0
NC = 2
NS = 16
L = 16
NW = NC * NS
RPW = B // NW
NGROUP = RPW // L

_LN2 = 0.6931471805599453
_SQRT2 = 1.4142135623730951


NCHUNK = 4
CW = RPW // NCHUNK
GPC = CW // L
NACC = 8


def _ldam_body(xt_hbm, t_hbm, m_hbm, out_hbm,
               xb0, xb1, xb2, xb3, tv, mv, accv,
               sem0, sem1, sem2, sem3):
    wid = lax.axis_index("s") * NC + lax.axis_index("c")
    base = wid * RPW
    xbufs = (xb0, xb1, xb2, xb3)
    sems = (sem0, sem1, sem2, sem3)
    copies = [
        pltpu.async_copy(
            xt_hbm.at[:, pl.ds(base + kc * CW, CW)], xbufs[kc], sems[kc])
        for kc in range(NCHUNK)
    ]
    pltpu.sync_copy(t_hbm.at[pl.ds(base, RPW)], tv)
    pltpu.sync_copy(m_hbm, mv.at[pl.ds(0, C)])

    zero = jnp.zeros((L,), jnp.float32)

    def make_group_body(xv, kc):
        def group_body(g, acc):
            mreg = [mv[pl.ds(16 * k, 16)] for k in range(7)]
            r0 = g * L
            tt = tv[pl.ds(kc * CW + r0, L)]
            mxs = [xv[c, pl.ds(r0, L)] for c in range(NACC)]
            for c in range(NACC, C):
                mxs[c % NACC] = jnp.maximum(mxs[c % NACC],
                                            xv[c, pl.ds(r0, L)])
            mx = mxs[0]
            for a in range(1, NACC):
                mx = jnp.maximum(mx, mxs[a])
            kk = S_SCALE * mx
            sss = [zero] * NACC
            zts = [zero] * NACC
            for c in range(C):
                v = xv[c, pl.ds(r0, L)]
                mc = mreg[c >> 4][c & 15]
                sel = tt == c
                w = S_SCALE * jnp.where(sel, v - mc, v)
                sss[c % NACC] = sss[c % NACC] + jnp.exp(w - kk)
                zts[c % NACC] = jnp.where(sel, w, zts[c % NACC])
            ss = sss[0]
            ztm = zts[0]
            for a in range(1, NACC):
                ss = ss + sss[a]
                ztm = ztm + zts[a]
            bits = lax.bitcast_convert_type(ss, jnp.int32)
            ex = lax.shift_right_arithmetic(bits, 23) - 127
            mf = lax.bitcast_convert_type(
                lax.bitwise_or(lax.bitwise_and(bits, 0x7FFFFF), 0x3F800000),
                jnp.float32)
            big = mf > _SQRT2
            mf = jnp.where(big, mf * 0.5, mf)
            ex = jnp.where(big, ex + 1, ex)
            u = (mf - 1.0) / (mf + 1.0)
            u2 = u * u
            ln = ex.astype(jnp.float32) * _LN2 + 2.0 * u * (
                1.0 + u2 * (0.3333333333 + u2 * 0.2))
            nll = kk + ln - ztm
            return acc + nll * (1.0 / B)

        return group_body

    acc = zero
    for kc in range(NCHUNK):
        copies[kc].wait()
        acc = lax.fori_loop(0, GPC, make_group_body(xbufs[kc], kc), acc)
    accv[...] = acc
    pltpu.sync_copy(accv, out_hbm.at[wid])


_ldam_sc = functools.partial(
    pl.kernel,
    out_type=jax.ShapeDtypeStruct((NW, L), jnp.float32),
    mesh=plsc.VectorSubcoreMesh(core_axis_name="c", subcore_axis_name="s"),
    compiler_params=pltpu.CompilerParams(use_tc_tiling_on_sc=True),
    scratch_types=[
        pltpu.VMEM((C, CW), jnp.float32),
        pltpu.VMEM((C, CW), jnp.float32),
        pltpu.VMEM((C, CW), jnp.float32),
        pltpu.VMEM((C, CW), jnp.float32),
        pltpu.VMEM((RPW,), jnp.int32),
        pltpu.VMEM((112,), jnp.float32),
        pltpu.VMEM((L,), jnp.float32),
        pltpu.SemaphoreType.DMA,
        pltpu.SemaphoreType.DMA,
        pltpu.SemaphoreType.DMA,
        pltpu.SemaphoreType.DMA,
    ],
)(_ldam_body)


def _sum_body(p_ref, o_ref):
    o_ref[0, 0] = jnp.sum(p_ref[...])


_sum_tc = pl.pallas_call(
    _sum_body,
    out_shape=jax.ShapeDtypeStruct((1, 1), jnp.float32),
    out_specs=pl.BlockSpec(memory_space=pltpu.SMEM),
)


def kernel(x, target, m_list):
    parts = _ldam_sc(x.T, target, m_list)
    return _sum_tc(parts)[0, 0]


# speedup vs baseline: 1.0029x; 1.0029x over previous
import functools

import jax
import jax.numpy as jnp
from jax import lax
from jax.experimental import pallas as pl
from jax.experimental.pallas import tpu as pltpu
from jax.experimental.pallas import tpu_sc as plsc

B = 16384
C = 100
S_SCALE = 30.0
NC = 2
NS = 16
L = 16
NW = NC * NS
RPW = B // NW
NGROUP = RPW // L
NACC = 8

_LN2 = 0.6931471805599453
_SQRT2 = 1.4142135623730951

_GDN = lax.GatherDimensionNumbers(
    offset_dims=(), collapsed_slice_dims=(0,), start_index_map=(0,))


def _vgather16(vec, idx):
    return lax.gather(vec, idx[:, None], dimension_numbers=_GDN,
                      slice_sizes=(1,),
                      mode=lax.GatherScatterMode.PROMISE_IN_BOUNDS)


def _ldam_body(xf_hbm, t_hbm, m_hbm, out_hbm,
               slab, tv, mvv, accv, sem_slab):
    wid = lax.axis_index("s") * NC + lax.axis_index("c")
    base = wid * RPW
    col_copies = [
        pltpu.async_copy(
            xf_hbm.at[pl.ds(c * B + base, RPW)],
            slab.at[pl.ds(c * RPW, RPW)], sem_slab)
        for c in range(C)
    ]
    pltpu.sync_copy(t_hbm.at[pl.ds(base, RPW)], tv)
    pltpu.sync_copy(m_hbm, mvv.at[pl.ds(0, C)])

    zero = jnp.zeros((L,), jnp.float32)
    iot = lax.iota(jnp.int32, L)
    lane_masks = [iot == i for i in range(L)]
    mreg = [mvv[pl.ds(16 * k, 16)] for k in range(7)]

    for cp in col_copies:
        cp.wait()

    def margin_body(j, zt):
        r0 = j * L
        tt = tv[pl.ds(r0, L)]
        low = lax.bitwise_and(tt, 15)
        hi = lax.shift_right_logical(tt, 4)
        mt = _vgather16(mreg[0], low)
        for k in range(1, 7):
            mt = jnp.where(hi == k, _vgather16(mreg[k], low), mt)
        for i in range(L):
            off = tt[i] * RPW + r0
            v = slab[pl.ds(off, L)]
            v2 = jnp.where(lane_masks[i], v - mt, v)
            zt = zt + jnp.where(lane_masks[i], v2, 0.0)
            slab[pl.ds(off, L)] = v2
        return zt

    ztacc = lax.fori_loop(0, NGROUP, margin_body, zero)

    def group_body(g, acc):
        r0 = g * L
        mxs = [slab[pl.ds(c * RPW + r0, L)] for c in range(NACC)]
        for c in range(NACC, C):
            mxs[c % NACC] = jnp.maximum(mxs[c % NACC],
                                        slab[pl.ds(c * RPW + r0, L)])
        mx = mxs[0]
        for a in range(1, NACC):
            mx = jnp.maximum(mx, mxs[a])
        kk = S_SCALE * mx
        sss = [zero] * NACC
        for c in range(C):
            v = slab[pl.ds(c * RPW + r0, L)]
            sss[c % NACC] = sss[c % NACC] + jnp.exp(S_SCALE * v - kk)
        ss = sss[0]
        for a in range(1, NACC):
            ss = ss + sss[a]
        bits = lax.bitcast_convert_type(ss, jnp.int32)
        ex = lax.shift_right_arithmetic(bits, 23) - 127
        mf = lax.bitcast_convert_type(
            lax.bitwise_or(lax.bitwise_and(bits, 0x7FFFFF), 0x3F800000),
            jnp.float32)
        big = mf > _SQRT2
        mf = jnp.where(big, mf * 0.5, mf)
        ex = jnp.where(big, ex + 1, ex)
        u = (mf - 1.0) / (mf + 1.0)
        u2 = u * u
        ln = ex.astype(jnp.float32) * _LN2 + 2.0 * u * (
            1.0 + u2 * (0.3333333333 + u2 * 0.2))
        return acc + (kk + ln)

    acc = lax.fori_loop(0, NGROUP, group_body, zero)

    accv[...] = (acc - S_SCALE * ztacc) * (1.0 / B)
    pltpu.sync_copy(accv, out_hbm.at[wid])


_ldam_sc = functools.partial(
    pl.kernel,
    out_type=jax.ShapeDtypeStruct((NW, L), jnp.float32),
    mesh=plsc.VectorSubcoreMesh(core_axis_name="c", subcore_axis_name="s"),
    compiler_params=pltpu.CompilerParams(use_tc_tiling_on_sc=True),
    scratch_types=[
        pltpu.VMEM((C * RPW,), jnp.float32),
        pltpu.VMEM((RPW,), jnp.int32),
        pltpu.VMEM((112,), jnp.float32),
        pltpu.VMEM((L,), jnp.float32),
        pltpu.SemaphoreType.DMA,
    ],
)(_ldam_body)


def _sum_body(p_ref, o_ref):
    o_ref[0, 0] = jnp.sum(p_ref[...])


_sum_tc = pl.pallas_call(
    _sum_body,
    out_shape=jax.ShapeDtypeStruct((1, 1), jnp.float32),
    out_specs=pl.BlockSpec(memory_space=pltpu.SMEM),
)


def kernel(x, target, m_list):
    parts = _ldam_sc(x.T.reshape(-1), target, m_list)
    return _sum_tc(parts)[0, 0]
